# baseline (speedup 1.0000x reference)
import jax
import jax.numpy as jnp
from jax.experimental import pallas as pl
from jax.experimental.pallas import tpu as pltpu

_N_E = 8192
_E_DIM = 64
_RADIAL_BINS = 16
_ANGULAR_BINS = 512
_MAX_RADIUS = 18.0
_BETA = 0.25

_BLK = 2048


def _acosh(x):
    return jnp.log(x + jnp.sqrt(x - 1.0) * jnp.sqrt(x + 1.0))


def _vq_body(flat_ref, rc_ref, a_ref,
             z_ref, loss_ref, perp_ref, usage_ref, emean_ref,
             counts_acc, loss_acc):
    pid = pl.program_id(0)
    nb = pl.num_programs(0)
    n_total = nb * _BLK

    @pl.when(pid == 0)
    def _init():
        counts_acc[...] = jnp.zeros_like(counts_acc)
        loss_acc[...] = jnp.zeros_like(loss_acc)

    ii = jax.lax.broadcasted_iota(jnp.int32, (_E_DIM, _E_DIM), 0)
    jj = jax.lax.broadcasted_iota(jnp.int32, (_E_DIM, _E_DIM), 1)
    eye_b = (ii == jj).astype(jnp.bfloat16)
    eye_f = (ii == jj).astype(jnp.float32)

    fb = flat_ref[...]
    hi = fb.astype(jnp.bfloat16)
    r1 = fb - hi.astype(jnp.float32)
    mid = r1.astype(jnp.bfloat16)
    lo = (r1 - mid.astype(jnp.float32)).astype(jnp.bfloat16)

    def _tr(part):
        return jax.lax.dot_general(eye_b, part, (((1,), (1,)), ((), ())),
                                   preferred_element_type=jnp.float32)

    flatb = _tr(hi) + _tr(mid) + _tr(lo)

    u_time = flatb[0:1, :]
    row = jax.lax.broadcasted_iota(jnp.int32, (_E_DIM, 1), 0)
    space = jnp.where(row == 0, 0.0, flatb)

    nrm = jnp.sqrt(jnp.sum(space * space, axis=0, keepdims=True))
    w = space / jnp.maximum(nrm, 1e-12)

    sim = jax.lax.dot_general(a_ref[...], w,
                              (((1,), (0,)), ((), ())),
                              preferred_element_type=jnp.float32)
    ms = jnp.max(sim, axis=0, keepdims=True)
    onehot_w = (sim == ms).astype(jnp.float32)
    w_hard = jax.lax.dot_general(a_ref[...], onehot_w,
                                 (((0,), (0,)), ((), ())),
                                 preferred_element_type=jnp.float32)

    r = _acosh(jnp.maximum(u_time, 1.0 + 1e-7))
    rc_c = jnp.clip(rc_ref[...][:, 0:1], 0.01, _MAX_RADIUS)
    dr = (r - rc_c) ** 2
    mr = jnp.min(dr, axis=0, keepdims=True)
    onehot_r = (dr == mr).astype(jnp.float32)
    r_hard = jnp.sum(onehot_r * rc_c, axis=0, keepdims=True)

    cnt = jax.lax.dot_general(onehot_r, onehot_w,
                              (((1,), (1,)), ((), ())),
                              preferred_element_type=jnp.float32)
    counts_acc[...] += cnt

    scale = jnp.tanh(r_hard * 0.5)
    xq = scale * w_hard
    x2 = jnp.sum(xq * xq, axis=0, keepdims=True)
    denom = jnp.maximum(1.0 - x2, 1e-7)
    xsp = (2.0 / denom) * xq
    t2 = jnp.sqrt(1.0 + jnp.sum(xsp * xsp, axis=0, keepdims=True))

    inner = jnp.sum(flatb * xsp, axis=0, keepdims=True) - u_time * t2
    dist = _acosh(jnp.maximum(-inner, 1.0 + 1e-7))
    loss_acc[...] += jnp.sum(dist).reshape(1, 1)

    xq_full = jnp.where(row == 0, t2, xsp)
    z_ref[...] = jax.lax.dot_general(xq_full, eye_f,
                                     (((0,), (0,)), ((), ())),
                                     precision=jax.lax.Precision.HIGHEST,
                                     preferred_element_type=jnp.float32)

    @pl.when(pid == nb - 1)
    def _fin():
        e_mean = counts_acc[...] / jnp.float32(n_total)
        emean_ref[...] = e_mean
        ent = jnp.sum(e_mean * jnp.log(e_mean + 1e-10))
        perp_ref[...] = jnp.exp(-ent).reshape(1, 1)
        usage_ref[...] = (jnp.sum((e_mean > 0).astype(jnp.float32))
                          / jnp.float32(_N_E)).reshape(1, 1)
        loss_ref[...] = _BETA * loss_acc[...] / jnp.float32(n_total)


def kernel(u_hyp, r_centres, angular_weight):
    u_shape = u_hyp.shape
    flat = u_hyp.reshape(-1, _E_DIM)
    n = flat.shape[0]
    grid = n // _BLK

    a64 = jnp.concatenate(
        [jnp.zeros((_ANGULAR_BINS, 1), angular_weight.dtype), angular_weight],
        axis=1)
    rc_rep = jnp.broadcast_to(r_centres.reshape(_RADIAL_BINS, 1),
                              (_RADIAL_BINS, 128))

    z, loss, perp, usage, emean = pl.pallas_call(
        _vq_body,
        grid=(grid,),
        in_specs=[
            pl.BlockSpec((_BLK, _E_DIM), lambda i: (i, 0)),
            pl.BlockSpec((_RADIAL_BINS, 128), lambda i: (0, 0)),
            pl.BlockSpec((_ANGULAR_BINS, _E_DIM), lambda i: (0, 0)),
        ],
        out_specs=[
            pl.BlockSpec((_BLK, _E_DIM), lambda i: (i, 0)),
            pl.BlockSpec((1, 1), lambda i: (0, 0)),
            pl.BlockSpec((1, 1), lambda i: (0, 0)),
            pl.BlockSpec((1, 1), lambda i: (0, 0)),
            pl.BlockSpec((_RADIAL_BINS, _ANGULAR_BINS), lambda i: (0, 0)),
        ],
        out_shape=[
            jax.ShapeDtypeStruct((n, _E_DIM), jnp.float32),
            jax.ShapeDtypeStruct((1, 1), jnp.float32),
            jax.ShapeDtypeStruct((1, 1), jnp.float32),
            jax.ShapeDtypeStruct((1, 1), jnp.float32),
            jax.ShapeDtypeStruct((_RADIAL_BINS, _ANGULAR_BINS), jnp.float32),
        ],
        scratch_shapes=[
            pltpu.VMEM((_RADIAL_BINS, _ANGULAR_BINS), jnp.float32),
            pltpu.VMEM((1, 1), jnp.float32),
        ],
    )(flat, rc_rep, a64)

    return (loss[0, 0], z.reshape(u_shape), perp[0, 0], usage[0, 0],
            emean.reshape(-1))


# speedup vs baseline: 2.0872x; 2.0872x over previous
import jax
import jax.numpy as jnp
from jax.experimental import pallas as pl
from jax.experimental.pallas import tpu as pltpu

_N_E = 8192
_E_DIM = 64
_RADIAL_BINS = 16
_ANGULAR_BINS = 512
_MAX_RADIUS = 18.0
_BETA = 0.25


def _acosh(x):
    return jnp.log(x + jnp.sqrt(x - 1.0) * jnp.sqrt(x + 1.0))


def _vq_body(n_total, flatt_ref, rc_ref, a_ref,
             z_ref, loss_ref, perp_ref, usage_ref, emean_ref,
             counts_acc, loss_acc):
    pid = pl.program_id(0)
    nb = pl.num_programs(0)

    @pl.when(pid == 0)
    def _init():
        counts_acc[...] = jnp.zeros_like(counts_acc)
        loss_acc[...] = jnp.zeros_like(loss_acc)

    flatb = flatt_ref[0]
    u_time = flatb[0:1, :]
    row = jax.lax.broadcasted_iota(jnp.int32, (_E_DIM, 1), 0)
    space = jnp.where(row == 0, 0.0, flatb)

    nrm = jnp.sqrt(jnp.sum(space * space, axis=0, keepdims=True))
    w = space / jnp.maximum(nrm, 1e-12)

    sim = jax.lax.dot_general(a_ref[...], w,
                              (((1,), (0,)), ((), ())),
                              preferred_element_type=jnp.float32)
    ms = jnp.max(sim, axis=0, keepdims=True)
    onehot_w = (sim == ms).astype(jnp.float32)
    w_hard = jax.lax.dot_general(a_ref[...], onehot_w,
                                 (((0,), (0,)), ((), ())),
                                 preferred_element_type=jnp.float32)

    r = _acosh(jnp.maximum(u_time, 1.0 + 1e-7))
    rc_c = jnp.clip(rc_ref[...][:, 0:1], 0.01, _MAX_RADIUS)
    dr = (r - rc_c) ** 2
    mr = jnp.min(dr, axis=0, keepdims=True)
    onehot_r = (dr == mr).astype(jnp.float32)
    r_hard = jnp.sum(onehot_r * rc_c, axis=0, keepdims=True)

    cnt = jax.lax.dot_general(onehot_r, onehot_w,
                              (((1,), (1,)), ((), ())),
                              preferred_element_type=jnp.float32)
    counts_acc[...] += cnt

    scale = jnp.tanh(r_hard * 0.5)
    xq = scale * w_hard
    x2 = jnp.sum(xq * xq, axis=0, keepdims=True)
    denom = jnp.maximum(1.0 - x2, 1e-7)
    xsp = (2.0 / denom) * xq
    t2 = jnp.sqrt(1.0 + jnp.sum(xsp * xsp, axis=0, keepdims=True))

    inner = jnp.sum(flatb * xsp, axis=0, keepdims=True) - u_time * t2
    dist = _acosh(jnp.maximum(-inner, 1.0 + 1e-7))
    loss_acc[...] += jnp.sum(dist).reshape(1, 1)

    z_ref[0] = jnp.where(row == 0, t2, xsp)

    @pl.when(pid == nb - 1)
    def _fin():
        e_mean = counts_acc[...] / jnp.float32(n_total)
        emean_ref[...] = e_mean
        ent = jnp.sum(e_mean * jnp.log(e_mean + 1e-10))
        perp_ref[...] = jnp.exp(-ent).reshape(1, 1)
        usage_ref[...] = (jnp.sum((e_mean > 0).astype(jnp.float32))
                          / jnp.float32(_N_E)).reshape(1, 1)
        loss_ref[...] = _BETA * loss_acc[...] / jnp.float32(n_total)


def kernel(u_hyp, r_centres, angular_weight):
    n0, n1, d = u_hyp.shape
    n_total = n0 * n1

    flatt = jnp.transpose(u_hyp, (0, 2, 1))
    a64 = jnp.concatenate(
        [jnp.zeros((_ANGULAR_BINS, 1), angular_weight.dtype), angular_weight],
        axis=1)
    rc_rep = jnp.broadcast_to(r_centres.reshape(_RADIAL_BINS, 1),
                              (_RADIAL_BINS, 128))

    import functools
    zt, loss, perp, usage, emean = pl.pallas_call(
        functools.partial(_vq_body, n_total),
        grid=(n0,),
        in_specs=[
            pl.BlockSpec((1, d, n1), lambda i: (i, 0, 0)),
            pl.BlockSpec((_RADIAL_BINS, 128), lambda i: (0, 0)),
            pl.BlockSpec((_ANGULAR_BINS, _E_DIM), lambda i: (0, 0)),
        ],
        out_specs=[
            pl.BlockSpec((1, d, n1), lambda i: (i, 0, 0)),
            pl.BlockSpec((1, 1), lambda i: (0, 0)),
            pl.BlockSpec((1, 1), lambda i: (0, 0)),
            pl.BlockSpec((1, 1), lambda i: (0, 0)),
            pl.BlockSpec((_RADIAL_BINS, _ANGULAR_BINS), lambda i: (0, 0)),
        ],
        out_shape=[
            jax.ShapeDtypeStruct((n0, d, n1), jnp.float32),
            jax.ShapeDtypeStruct((1, 1), jnp.float32),
            jax.ShapeDtypeStruct((1, 1), jnp.float32),
            jax.ShapeDtypeStruct((1, 1), jnp.float32),
            jax.ShapeDtypeStruct((_RADIAL_BINS, _ANGULAR_BINS), jnp.float32),
        ],
        scratch_shapes=[
            pltpu.VMEM((_RADIAL_BINS, _ANGULAR_BINS), jnp.float32),
            pltpu.VMEM((1, 1), jnp.float32),
        ],
    )(flatt, rc_rep, a64)

    z = jnp.transpose(zt, (0, 2, 1))
    return (loss[0, 0], z, perp[0, 0], usage[0, 0], emean.reshape(-1))


# speedup vs baseline: 2.4309x; 1.1647x over previous
import jax
import jax.numpy as jnp
from jax.experimental import pallas as pl
from jax.experimental.pallas import tpu as pltpu

_N_E = 8192
_E_DIM = 64
_RADIAL_BINS = 16
_ANGULAR_BINS = 512
_MAX_RADIUS = 18.0
_BETA = 0.25


def _acosh(x):
    return jnp.log(x + jnp.sqrt(x - 1.0) * jnp.sqrt(x + 1.0))


def _vq_body(n_total, flatt_ref, rc_ref, a_ref,
             z_ref, loss_ref, perp_ref, usage_ref, emean_ref,
             counts_acc, loss_acc):
    pid = pl.program_id(0)
    nb = pl.num_programs(0)

    @pl.when(pid == 0)
    def _init():
        counts_acc[...] = jnp.zeros_like(counts_acc)
        loss_acc[...] = jnp.zeros_like(loss_acc)

    ns = flatt_ref.shape[0]
    if ns == 1:
        flatb = flatt_ref[0]
    else:
        flatb = jnp.concatenate([flatt_ref[s] for s in range(ns)], axis=1)
    u_time = flatb[0:1, :]
    row = jax.lax.broadcasted_iota(jnp.int32, (_E_DIM, 1), 0)
    space = jnp.where(row == 0, 0.0, flatb)

    nrm = jnp.sqrt(jnp.sum(space * space, axis=0, keepdims=True))
    w = space / jnp.maximum(nrm, 1e-12)

    sim = jax.lax.dot_general(a_ref[...], w,
                              (((1,), (0,)), ((), ())),
                              preferred_element_type=jnp.float32)
    ms = jnp.max(sim, axis=0, keepdims=True)
    onehot_w = (sim == ms).astype(jnp.float32)
    w_hard = jax.lax.dot_general(a_ref[...], onehot_w,
                                 (((0,), (0,)), ((), ())),
                                 preferred_element_type=jnp.float32)

    r = _acosh(jnp.maximum(u_time, 1.0 + 1e-7))
    rc_c = jnp.clip(rc_ref[...][:, 0:1], 0.01, _MAX_RADIUS)
    dr = (r - rc_c) ** 2
    mr = jnp.min(dr, axis=0, keepdims=True)
    onehot_r = (dr == mr).astype(jnp.float32)
    r_hard = jnp.sum(onehot_r * rc_c, axis=0, keepdims=True)

    cnt = jax.lax.dot_general(onehot_r, onehot_w,
                              (((1,), (1,)), ((), ())),
                              preferred_element_type=jnp.float32)
    counts_acc[...] += cnt

    scale = jnp.tanh(r_hard * 0.5)
    xq = scale * w_hard
    x2 = jnp.sum(xq * xq, axis=0, keepdims=True)
    denom = jnp.maximum(1.0 - x2, 1e-7)
    xsp = (2.0 / denom) * xq
    t2 = jnp.sqrt(1.0 + jnp.sum(xsp * xsp, axis=0, keepdims=True))

    inner = jnp.sum(flatb * xsp, axis=0, keepdims=True) - u_time * t2
    dist = _acosh(jnp.maximum(-inner, 1.0 + 1e-7))
    loss_acc[...] += jnp.sum(dist).reshape(1, 1)

    zout = jnp.where(row == 0, t2, xsp)
    bsz = zout.shape[1] // ns
    for s in range(ns):
        z_ref[s] = zout[:, s * bsz:(s + 1) * bsz]

    @pl.when(pid == nb - 1)
    def _fin():
        e_mean = counts_acc[...] / jnp.float32(n_total)
        emean_ref[...] = e_mean
        ent = jnp.sum(e_mean * jnp.log(e_mean + 1e-10))
        perp_ref[...] = jnp.exp(-ent).reshape(1, 1)
        usage_ref[...] = (jnp.sum((e_mean > 0).astype(jnp.float32))
                          / jnp.float32(_N_E)).reshape(1, 1)
        loss_ref[...] = _BETA * loss_acc[...] / jnp.float32(n_total)


def kernel(u_hyp, r_centres, angular_weight):
    n0, n1, d = u_hyp.shape
    n_total = n0 * n1

    flatt = jnp.transpose(u_hyp, (0, 2, 1))
    a64 = jnp.concatenate(
        [jnp.zeros((_ANGULAR_BINS, 1), angular_weight.dtype), angular_weight],
        axis=1)
    rc_rep = jnp.broadcast_to(r_centres.reshape(_RADIAL_BINS, 1),
                              (_RADIAL_BINS, 128))

    slabs = 2 if n0 % 2 == 0 else 1

    import functools
    zt, loss, perp, usage, emean = pl.pallas_call(
        functools.partial(_vq_body, n_total),
        grid=(n0 // slabs,),
        in_specs=[
            pl.BlockSpec((slabs, d, n1), lambda i: (i, 0, 0)),
            pl.BlockSpec((_RADIAL_BINS, 128), lambda i: (0, 0)),
            pl.BlockSpec((_ANGULAR_BINS, _E_DIM), lambda i: (0, 0)),
        ],
        out_specs=[
            pl.BlockSpec((slabs, d, n1), lambda i: (i, 0, 0)),
            pl.BlockSpec((1, 1), lambda i: (0, 0)),
            pl.BlockSpec((1, 1), lambda i: (0, 0)),
            pl.BlockSpec((1, 1), lambda i: (0, 0)),
            pl.BlockSpec((_RADIAL_BINS, _ANGULAR_BINS), lambda i: (0, 0)),
        ],
        out_shape=[
            jax.ShapeDtypeStruct((n0, d, n1), jnp.float32),
            jax.ShapeDtypeStruct((1, 1), jnp.float32),
            jax.ShapeDtypeStruct((1, 1), jnp.float32),
            jax.ShapeDtypeStruct((1, 1), jnp.float32),
            jax.ShapeDtypeStruct((_RADIAL_BINS, _ANGULAR_BINS), jnp.float32),
        ],
        scratch_shapes=[
            pltpu.VMEM((_RADIAL_BINS, _ANGULAR_BINS), jnp.float32),
            pltpu.VMEM((1, 1), jnp.float32),
        ],
    )(flatt, rc_rep, a64)

    z = jnp.transpose(zt, (0, 2, 1))
    return (loss[0, 0], z, perp[0, 0], usage[0, 0], emean.reshape(-1))


# speedup vs baseline: 2.4753x; 1.0183x over previous
import jax
import jax.numpy as jnp
from jax.experimental import pallas as pl
from jax.experimental.pallas import tpu as pltpu

_N_E = 8192
_E_DIM = 64
_RADIAL_BINS = 16
_ANGULAR_BINS = 512
_MAX_RADIUS = 18.0
_BETA = 0.25


def _acosh(x):
    return jnp.log(x + jnp.sqrt(x - 1.0) * jnp.sqrt(x + 1.0))


def _vq_body(n_total, flatt_ref, rc_ref, a_ref,
             z_ref, loss_ref, perp_ref, usage_ref, emean_ref,
             counts_acc, loss_acc):
    pid = pl.program_id(0)
    nb = pl.num_programs(0)

    @pl.when(pid == 0)
    def _init():
        counts_acc[...] = jnp.zeros_like(counts_acc)
        loss_acc[...] = jnp.zeros_like(loss_acc)

    ns = flatt_ref.shape[0]
    if ns == 1:
        flatb = flatt_ref[0]
    else:
        flatb = jnp.concatenate([flatt_ref[s] for s in range(ns)], axis=1)
    u_time = flatb[0:1, :]
    row = jax.lax.broadcasted_iota(jnp.int32, (_E_DIM, 1), 0)
    space = jnp.where(row == 0, 0.0, flatb)

    nrm = jnp.sqrt(jnp.sum(space * space, axis=0, keepdims=True))
    w = space / jnp.maximum(nrm, 1e-12)

    sim = jax.lax.dot_general(a_ref[...], w,
                              (((1,), (0,)), ((), ())),
                              preferred_element_type=jnp.float32)
    ms = jnp.max(sim, axis=0, keepdims=True)
    onehot_w = (sim == ms).astype(jnp.float32)
    w_hard = jax.lax.dot_general(a_ref[...], onehot_w,
                                 (((0,), (0,)), ((), ())),
                                 preferred_element_type=jnp.float32)

    r = _acosh(jnp.maximum(u_time, 1.0 + 1e-7))
    rc_c = jnp.clip(rc_ref[...][:, 0:1], 0.01, _MAX_RADIUS)
    dr = (r - rc_c) ** 2
    mr = jnp.min(dr, axis=0, keepdims=True)
    onehot_r = (dr == mr).astype(jnp.float32)
    r_hard = jnp.sum(onehot_r * rc_c, axis=0, keepdims=True)

    cnt = jax.lax.dot_general(onehot_r, onehot_w,
                              (((1,), (1,)), ((), ())),
                              preferred_element_type=jnp.float32)
    counts_acc[...] += cnt

    scale = jnp.tanh(r_hard * 0.5)
    xq = scale * w_hard
    x2 = jnp.sum(xq * xq, axis=0, keepdims=True)
    denom = jnp.maximum(1.0 - x2, 1e-7)
    xsp = (2.0 / denom) * xq
    t2 = jnp.sqrt(1.0 + jnp.sum(xsp * xsp, axis=0, keepdims=True))

    inner = jnp.sum(flatb * xsp, axis=0, keepdims=True) - u_time * t2
    dist = _acosh(jnp.maximum(-inner, 1.0 + 1e-7))
    loss_acc[...] += jnp.sum(dist).reshape(1, 1)

    zout = jnp.where(row == 0, t2, xsp)
    bsz = zout.shape[1] // ns
    for s in range(ns):
        z_ref[s] = zout[:, s * bsz:(s + 1) * bsz]

    @pl.when(pid == nb - 1)
    def _fin():
        e_mean = counts_acc[...] / jnp.float32(n_total)
        emean_ref[...] = e_mean
        ent = jnp.sum(e_mean * jnp.log(e_mean + 1e-10))
        perp_ref[...] = jnp.exp(-ent).reshape(1, 1)
        usage_ref[...] = (jnp.sum((e_mean > 0).astype(jnp.float32))
                          / jnp.float32(_N_E)).reshape(1, 1)
        loss_ref[...] = _BETA * loss_acc[...] / jnp.float32(n_total)


def kernel(u_hyp, r_centres, angular_weight):
    n0, n1, d = u_hyp.shape
    n_total = n0 * n1

    flatt = jnp.transpose(u_hyp, (0, 2, 1))
    a64 = jnp.concatenate(
        [jnp.zeros((_ANGULAR_BINS, 1), angular_weight.dtype), angular_weight],
        axis=1)
    rc_rep = jnp.broadcast_to(r_centres.reshape(_RADIAL_BINS, 1),
                              (_RADIAL_BINS, 128))

    slabs = 4 if n0 % 4 == 0 else (2 if n0 % 2 == 0 else 1)

    import functools
    zt, loss, perp, usage, emean = pl.pallas_call(
        functools.partial(_vq_body, n_total),
        grid=(n0 // slabs,),
        in_specs=[
            pl.BlockSpec((slabs, d, n1), lambda i: (i, 0, 0)),
            pl.BlockSpec((_RADIAL_BINS, 128), lambda i: (0, 0)),
            pl.BlockSpec((_ANGULAR_BINS, _E_DIM), lambda i: (0, 0)),
        ],
        out_specs=[
            pl.BlockSpec((slabs, d, n1), lambda i: (i, 0, 0)),
            pl.BlockSpec((1, 1), lambda i: (0, 0)),
            pl.BlockSpec((1, 1), lambda i: (0, 0)),
            pl.BlockSpec((1, 1), lambda i: (0, 0)),
            pl.BlockSpec((_RADIAL_BINS, _ANGULAR_BINS), lambda i: (0, 0)),
        ],
        out_shape=[
            jax.ShapeDtypeStruct((n0, d, n1), jnp.float32),
            jax.ShapeDtypeStruct((1, 1), jnp.float32),
            jax.ShapeDtypeStruct((1, 1), jnp.float32),
            jax.ShapeDtypeStruct((1, 1), jnp.float32),
            jax.ShapeDtypeStruct((_RADIAL_BINS, _ANGULAR_BINS), jnp.float32),
        ],
        scratch_shapes=[
            pltpu.VMEM((_RADIAL_BINS, _ANGULAR_BINS), jnp.float32),
            pltpu.VMEM((1, 1), jnp.float32),
        ],
    )(flatt, rc_rep, a64)

    z = jnp.transpose(zt, (0, 2, 1))
    return (loss[0, 0], z, perp[0, 0], usage[0, 0], emean.reshape(-1))
